# baseline (speedup 1.0000x reference)
import functools

import jax
import jax.numpy as jnp
from jax import lax
from jax.experimental import pallas as pl
from jax.experimental.pallas import tpu as pltpu
from jax.experimental.pallas import tpu_sc as plsc

_NBUF = 15
_CHUNK_ROWS = 8


@functools.cache
def _build(num_rows: int, row_start: int, table_rows: int, dim: int):
    info = plsc.get_sparse_core_info()
    nw = info.num_cores * info.num_subcores
    assert num_rows % (nw * _CHUNK_ROWS) == 0
    n_chunks = num_rows // (nw * _CHUNK_ROWS)
    mesh = plsc.VectorSubcoreMesh(core_axis_name="c", subcore_axis_name="s")

    @functools.partial(
        pl.kernel,
        out_type=jax.ShapeDtypeStruct((num_rows, dim), jnp.float32),
        mesh=mesh,
        scratch_types=[
            [pltpu.VMEM((_CHUNK_ROWS, dim), jnp.float32) for _ in range(_NBUF)],
            [pltpu.SemaphoreType.DMA for _ in range(_NBUF)],
            [pltpu.SemaphoreType.DMA for _ in range(_NBUF)],
        ],
    )
    def copy_kernel(table_hbm, out_hbm, bufs, rsems, wsems):
        wid = lax.axis_index("s") * info.num_cores + lax.axis_index("c")

        def chunk_row(i):
            return (wid + i * nw) * _CHUNK_ROWS

        def rd(i, b):
            src = table_hbm.at[pl.ds(row_start + chunk_row(i), _CHUNK_ROWS)]
            return pltpu.async_copy(src, bufs[b], rsems[b])

        def wr(i, b):
            dst = out_hbm.at[pl.ds(chunk_row(i), _CHUNK_ROWS)]
            return pltpu.async_copy(bufs[b], dst, wsems[b])

        reads = [None] * n_chunks
        writes = [None] * n_chunks
        for i in range(n_chunks):
            b = i % _NBUF
            if i >= _NBUF:
                writes[i - _NBUF].wait()
            reads[i] = rd(i, b)
            if i >= 8:
                reads[i - 8].wait()
                writes[i - 8] = wr(i - 8, (i - 8) % _NBUF)
        for i in range(max(0, n_chunks - 8), n_chunks):
            reads[i].wait()
            writes[i] = wr(i, i % _NBUF)
        for i in range(max(0, n_chunks - _NBUF), n_chunks):
            writes[i].wait()

    return copy_kernel


def kernel(input, weights):
    bsz, seq_len = input.shape
    table_rows, dim = weights.shape
    origin_shift = table_rows // 2
    start = int(-seq_len / 2)
    end = round(seq_len / 2 + 1e-05)
    num_rows = end - start
    row_start = origin_shift + start
    return _build(num_rows, row_start, table_rows, dim)(weights)


# speedup vs baseline: 1.0492x; 1.0492x over previous
import functools

import jax
import jax.numpy as jnp
from jax import lax
from jax.experimental import pallas as pl
from jax.experimental.pallas import tpu as pltpu
from jax.experimental.pallas import tpu_sc as plsc

_NBUF = 7
_CHUNK_ROWS = 16


@functools.cache
def _build(num_rows: int, row_start: int, table_rows: int, dim: int):
    info = plsc.get_sparse_core_info()
    nw = info.num_cores * info.num_subcores
    assert num_rows % (nw * _CHUNK_ROWS) == 0
    n_chunks = num_rows // (nw * _CHUNK_ROWS)
    mesh = plsc.VectorSubcoreMesh(core_axis_name="c", subcore_axis_name="s")

    @functools.partial(
        pl.kernel,
        out_type=jax.ShapeDtypeStruct((num_rows, dim), jnp.float32),
        mesh=mesh,
        scratch_types=[
            [pltpu.VMEM((_CHUNK_ROWS, dim), jnp.float32) for _ in range(_NBUF)],
            [pltpu.SemaphoreType.DMA for _ in range(_NBUF)],
            [pltpu.SemaphoreType.DMA for _ in range(_NBUF)],
        ],
    )
    def copy_kernel(table_hbm, out_hbm, bufs, rsems, wsems):
        wid = lax.axis_index("s") * info.num_cores + lax.axis_index("c")

        def chunk_row(i):
            return (wid + i * nw) * _CHUNK_ROWS

        def rd(i, b):
            src = table_hbm.at[pl.ds(row_start + chunk_row(i), _CHUNK_ROWS)]
            return pltpu.async_copy(src, bufs[b], rsems[b])

        def wr(i, b):
            dst = out_hbm.at[pl.ds(chunk_row(i), _CHUNK_ROWS)]
            return pltpu.async_copy(bufs[b], dst, wsems[b])

        reads = [None] * n_chunks
        writes = [None] * n_chunks
        for i in range(n_chunks):
            b = i % _NBUF
            if i >= _NBUF:
                writes[i - _NBUF].wait()
            reads[i] = rd(i, b)
            if i >= 5:
                reads[i - 5].wait()
                writes[i - 5] = wr(i - 5, (i - 5) % _NBUF)
        for i in range(max(0, n_chunks - 5), n_chunks):
            reads[i].wait()
            writes[i] = wr(i, i % _NBUF)
        for i in range(max(0, n_chunks - _NBUF), n_chunks):
            writes[i].wait()

    return copy_kernel


def kernel(input, weights):
    bsz, seq_len = input.shape
    table_rows, dim = weights.shape
    origin_shift = table_rows // 2
    start = int(-seq_len / 2)
    end = round(seq_len / 2 + 1e-05)
    num_rows = end - start
    row_start = origin_shift + start
    return _build(num_rows, row_start, table_rows, dim)(weights)
